# baseline (speedup 1.0000x reference)
import jax
import jax.numpy as jnp
from jax import lax
from jax.experimental import pallas as pl
from jax.experimental.pallas import tpu as pltpu
from jax.experimental.pallas import tpu_sc as plsc

N, C, H, W = 4, 96, 384, 384
P = H * W
NP = N * P
NW = 32
PPW = P // NW
CH = 128
CHUNKS = PPW // CH
G16 = CH // 16
CW = C // 32
TW = C // 2
OC = C


def _sc_body(table_hbm, gx_hbm, gy_hbm, out_hbm,
             gx_v, gy_v, idx_v, w_v, r_v, out_v,
             sem_gr, sem_g, sem_o):
    cid = lax.axis_index("c")
    sid = lax.axis_index("s")
    wid = sid * 2 + cid
    base = wid * PPW

    def start_grid(k, b):
        off = base + k * CH
        pltpu.make_async_copy(gx_hbm.at[pl.ds(off, CH)], gx_v.at[b], sem_gr.at[b]).start()
        pltpu.make_async_copy(gy_hbm.at[pl.ds(off, CH)], gy_v.at[b], sem_gr.at[b]).start()

    def wait_grid(k, b):
        off = base + k * CH
        pltpu.make_async_copy(gx_hbm.at[pl.ds(off, CH)], gx_v.at[b], sem_gr.at[b]).wait()
        pltpu.make_async_copy(gy_hbm.at[pl.ds(off, CH)], gy_v.at[b], sem_gr.at[b]).wait()

    def idx_compute(b):
        def idx_body(g, c2):
            s = pl.ds(g * 16, 16)
            x = gx_v[b, s]
            y = gy_v[b, s]
            ix = ((x + 1.0) * W - 1.0) * 0.5
            iy = ((y + 1.0) * H - 1.0) * 0.5
            ixt = ix.astype(jnp.int32)
            ixtf = ixt.astype(jnp.float32)
            mx = ix < ixtf
            ix0 = ixt - jnp.where(mx, 1, 0)
            fx0 = ixtf - jnp.where(mx, 1.0, 0.0)
            iyt = iy.astype(jnp.int32)
            iytf = iyt.astype(jnp.float32)
            my = iy < iytf
            iy0 = iyt - jnp.where(my, 1, 0)
            fy0 = iytf - jnp.where(my, 1.0, 0.0)
            wx1 = ix - fx0
            wx0 = 1.0 - wx1
            wy1 = iy - fy0
            wy0 = 1.0 - wy1
            vx0 = (ix0 >= 0) & (ix0 <= W - 1)
            vx1 = (ix0 >= -1) & (ix0 <= W - 2)
            vy0 = (iy0 >= 0) & (iy0 <= H - 1)
            vy1 = (iy0 >= -1) & (iy0 <= H - 2)
            wx0 = jnp.where(vx0, wx0, 0.0)
            wx1 = jnp.where(vx1, wx1, 0.0)
            wy0 = jnp.where(vy0, wy0, 0.0)
            wy1 = jnp.where(vy1, wy1, 0.0)
            cx0 = jnp.minimum(jnp.maximum(ix0, 0), W - 1)
            cx1 = jnp.minimum(jnp.maximum(ix0 + 1, 0), W - 1)
            cy0 = jnp.minimum(jnp.maximum(iy0, 0), H - 1)
            cy1 = jnp.minimum(jnp.maximum(iy0 + 1, 0), H - 1)
            rb0 = cy0 * W
            rb1 = cy1 * W
            idx_v[b, 0, s] = rb0 + cx0
            idx_v[b, 1, s] = rb0 + cx1
            idx_v[b, 2, s] = rb1 + cx0
            idx_v[b, 3, s] = rb1 + cx1
            w_v[b, 0, s] = wy0 * wx0
            w_v[b, 1, s] = wy0 * wx1
            w_v[b, 2, s] = wy1 * wx0
            w_v[b, 3, s] = wy1 * wx1
            return c2

        lax.fori_loop(0, G16, idx_body, 0)

    def start_gathers(b):
        for q in range(4):
            pltpu.make_async_copy(table_hbm.at[idx_v.at[b, q]], r_v.at[b, q],
                                  sem_g.at[b]).start()

    def wait_gathers(b):
        for q in range(4):
            pltpu.make_async_copy(table_hbm.at[idx_v.at[b, q]], r_v.at[b, q],
                                  sem_g.at[b]).wait()

    def combine(b):
        def cmb_body(g, c2):
            s = pl.ds(g * 16, 16)
            w00g = w_v[b, 0, s]
            w01g = w_v[b, 1, s]
            w10g = w_v[b, 2, s]
            w11g = w_v[b, 3, s]
            p0 = g * 16
            for i in range(16):
                px = p0 + i
                ws = (w00g[i], w01g[i], w10g[i], w11g[i])
                for j in range(CW):
                    cs = pl.ds(j * 16, 16)
                    acc_lo = None
                    acc_hi = None
                    for q in range(4):
                        wq = r_v[b, q, px, cs]
                        loq = lax.bitcast_convert_type(wq << 16, jnp.float32)
                        hiq = lax.bitcast_convert_type(wq, jnp.float32)
                        if acc_lo is None:
                            acc_lo = loq * ws[q]
                            acc_hi = hiq * ws[q]
                        else:
                            acc_lo = acc_lo + loq * ws[q]
                            acc_hi = acc_hi + hiq * ws[q]
                    out_v[b, px, pl.ds(j * 32, 16)] = acc_lo
                    out_v[b, px, pl.ds(j * 32 + 16, 16)] = acc_hi
            return c2

        lax.fori_loop(0, G16, cmb_body, 0)

    def start_out(k, b):
        off = base + k * CH
        pltpu.make_async_copy(out_v.at[b], out_hbm.at[pl.ds(off, CH)], sem_o.at[b]).start()

    def wait_out(k, b):
        off = base + k * CH
        pltpu.make_async_copy(out_v.at[b], out_hbm.at[pl.ds(off, CH)], sem_o.at[b]).wait()

    def step(k, b):
        def prefetch():
            wait_grid(k + 1, 1 - b)
            idx_compute(1 - b)
            start_gathers(1 - b)

        pl.when(k + 1 < CHUNKS)(prefetch)
        pl.when(k + 2 < CHUNKS)(lambda: start_grid(k + 2, b))
        wait_gathers(b)
        pl.when(k >= 2)(lambda: wait_out(k - 2, b))
        combine(b)
        start_out(k, b)

    start_grid(0, 0)
    wait_grid(0, 0)
    idx_compute(0)
    start_gathers(0)
    start_grid(1, 1)

    def loop_body(k2, carry):
        step(2 * k2, 0)
        step(2 * k2 + 1, 1)
        return carry

    lax.fori_loop(0, CHUNKS // 2, loop_body, 0)

    wait_out(CHUNKS - 2, 0)
    wait_out(CHUNKS - 1, 1)


def _make_sc_call():
    mesh = plsc.VectorSubcoreMesh(core_axis_name="c", subcore_axis_name="s")
    return pl.kernel(
        _sc_body,
        out_type=jax.ShapeDtypeStruct((P, OC), jnp.float32),
        mesh=mesh,
        scratch_types=[
            pltpu.VMEM((2, CH), jnp.float32),
            pltpu.VMEM((2, CH), jnp.float32),
            pltpu.VMEM((2, 4, CH), jnp.int32),
            pltpu.VMEM((2, 4, CH), jnp.float32),
            pltpu.VMEM((2, 4, CH, TW), jnp.int32),
            pltpu.VMEM((2, CH, OC), jnp.float32),
            pltpu.SemaphoreType.DMA((2,)),
            pltpu.SemaphoreType.DMA((2,)),
            pltpu.SemaphoreType.DMA((2,)),
        ],
        compiler_params=pltpu.CompilerParams(use_tc_tiling_on_sc=False),
    )


@jax.jit
def kernel(input, grid):
    sc_call = _make_sc_call()
    outs = []
    for n in range(N):
        bf = jnp.transpose(input[n], (1, 2, 0)).astype(jnp.bfloat16)
        il = jnp.transpose(bf.reshape(P, CW, 2, 16), (0, 1, 3, 2))
        table = lax.bitcast_convert_type(il, jnp.int32).reshape(P, TW)
        gx = grid[n, :, :, 0].reshape(P)
        gy = grid[n, :, :, 1].reshape(P)
        rows = sc_call(table, gx, gy)
        outs.append(jnp.transpose(rows.reshape(H, W, C), (2, 0, 1)))
    return jnp.stack(outs)


# speedup vs baseline: 1.2258x; 1.2258x over previous
import jax
import jax.numpy as jnp
from jax import lax
from jax.experimental import pallas as pl
from jax.experimental.pallas import tpu as pltpu
from jax.experimental.pallas import tpu_sc as plsc

N, C, H, W = 4, 96, 384, 384
P = H * W
NP = N * P
NW = 32
PPW = P // NW
CH = 96
CHUNKS = PPW // CH
G16 = CH // 16
CW = C // 32
TW = C // 2
OC = C


def _sc_body(table_hbm, gx_hbm, gy_hbm, out_hbm,
             gx_v, gy_v, idx_v, w_v, r_v, out_v,
             sem_gr, sem_g, sem_o):
    cid = lax.axis_index("c")
    sid = lax.axis_index("s")
    wid = sid * 2 + cid
    base = wid * PPW

    def start_grid(k, b):
        off = base + k * CH
        pltpu.make_async_copy(gx_hbm.at[pl.ds(off, CH)], gx_v.at[b], sem_gr.at[b]).start()
        pltpu.make_async_copy(gy_hbm.at[pl.ds(off, CH)], gy_v.at[b], sem_gr.at[b]).start()

    def wait_grid(k, b):
        off = base + k * CH
        pltpu.make_async_copy(gx_hbm.at[pl.ds(off, CH)], gx_v.at[b], sem_gr.at[b]).wait()
        pltpu.make_async_copy(gy_hbm.at[pl.ds(off, CH)], gy_v.at[b], sem_gr.at[b]).wait()

    def idx_compute(b):
        def idx_body(g, c2):
            s = pl.ds(g * 16, 16)
            x = gx_v[b, s]
            y = gy_v[b, s]
            ix = ((x + 1.0) * W - 1.0) * 0.5
            iy = ((y + 1.0) * H - 1.0) * 0.5
            ixt = ix.astype(jnp.int32)
            ixtf = ixt.astype(jnp.float32)
            mx = ix < ixtf
            ix0 = ixt - jnp.where(mx, 1, 0)
            fx0 = ixtf - jnp.where(mx, 1.0, 0.0)
            iyt = iy.astype(jnp.int32)
            iytf = iyt.astype(jnp.float32)
            my = iy < iytf
            iy0 = iyt - jnp.where(my, 1, 0)
            fy0 = iytf - jnp.where(my, 1.0, 0.0)
            wx1 = ix - fx0
            wx0 = 1.0 - wx1
            wy1 = iy - fy0
            wy0 = 1.0 - wy1
            vx0 = (ix0 >= 0) & (ix0 <= W - 1)
            vx1 = (ix0 >= -1) & (ix0 <= W - 2)
            vy0 = (iy0 >= 0) & (iy0 <= H - 1)
            vy1 = (iy0 >= -1) & (iy0 <= H - 2)
            wx0 = jnp.where(vx0, wx0, 0.0)
            wx1 = jnp.where(vx1, wx1, 0.0)
            wy0 = jnp.where(vy0, wy0, 0.0)
            wy1 = jnp.where(vy1, wy1, 0.0)
            cx0 = jnp.minimum(jnp.maximum(ix0, 0), W - 1)
            cx1 = jnp.minimum(jnp.maximum(ix0 + 1, 0), W - 1)
            cy0 = jnp.minimum(jnp.maximum(iy0, 0), H - 1)
            cy1 = jnp.minimum(jnp.maximum(iy0 + 1, 0), H - 1)
            rb0 = cy0 * W
            rb1 = cy1 * W
            idx_v[b, 0, s] = rb0 + cx0
            idx_v[b, 1, s] = rb0 + cx1
            idx_v[b, 2, s] = rb1 + cx0
            idx_v[b, 3, s] = rb1 + cx1
            w_v[b, 0, s] = wy0 * wx0
            w_v[b, 1, s] = wy0 * wx1
            w_v[b, 2, s] = wy1 * wx0
            w_v[b, 3, s] = wy1 * wx1
            return c2

        lax.fori_loop(0, G16, idx_body, 0)

    def start_gathers(b):
        for q in range(4):
            pltpu.make_async_copy(table_hbm.at[idx_v.at[b, q]], r_v.at[b, q],
                                  sem_g.at[b]).start()

    def wait_gathers(b):
        for q in range(4):
            pltpu.make_async_copy(table_hbm.at[idx_v.at[b, q]], r_v.at[b, q],
                                  sem_g.at[b]).wait()

    def combine(b):
        def cmb_body(g, c2):
            s = pl.ds(g * 16, 16)
            w00g = w_v[b, 0, s]
            w01g = w_v[b, 1, s]
            w10g = w_v[b, 2, s]
            w11g = w_v[b, 3, s]
            p0 = g * 16
            for i in range(16):
                px = p0 + i
                ws = (w00g[i], w01g[i], w10g[i], w11g[i])
                for j in range(C // 16):
                    cs = pl.ds(j * 16, 16)
                    acc = (r_v[b, 0, px, cs] * ws[0]
                           + r_v[b, 1, px, cs] * ws[1]
                           + r_v[b, 2, px, cs] * ws[2]
                           + r_v[b, 3, px, cs] * ws[3])
                    out_v[b, px, cs] = acc
            return c2

        lax.fori_loop(0, G16, cmb_body, 0)

    def start_out(k, b):
        off = base + k * CH
        pltpu.make_async_copy(out_v.at[b], out_hbm.at[pl.ds(off, CH)], sem_o.at[b]).start()

    def wait_out(k, b):
        off = base + k * CH
        pltpu.make_async_copy(out_v.at[b], out_hbm.at[pl.ds(off, CH)], sem_o.at[b]).wait()

    def step(k, b):
        def prefetch():
            wait_grid(k + 1, 1 - b)
            idx_compute(1 - b)
            start_gathers(1 - b)

        pl.when(k + 1 < CHUNKS)(prefetch)
        pl.when(k + 2 < CHUNKS)(lambda: start_grid(k + 2, b))
        wait_gathers(b)
        pl.when(k >= 2)(lambda: wait_out(k - 2, b))
        combine(b)
        start_out(k, b)

    start_grid(0, 0)
    wait_grid(0, 0)
    idx_compute(0)
    start_gathers(0)
    start_grid(1, 1)

    def loop_body(k2, carry):
        step(2 * k2, 0)
        step(2 * k2 + 1, 1)
        return carry

    lax.fori_loop(0, CHUNKS // 2, loop_body, 0)

    wait_out(CHUNKS - 2, 0)
    wait_out(CHUNKS - 1, 1)


def _make_sc_call():
    mesh = plsc.VectorSubcoreMesh(core_axis_name="c", subcore_axis_name="s")
    return pl.kernel(
        _sc_body,
        out_type=jax.ShapeDtypeStruct((P, OC), jnp.float32),
        mesh=mesh,
        scratch_types=[
            pltpu.VMEM((2, CH), jnp.float32),
            pltpu.VMEM((2, CH), jnp.float32),
            pltpu.VMEM((2, 4, CH), jnp.int32),
            pltpu.VMEM((2, 4, CH), jnp.float32),
            pltpu.VMEM((2, 4, CH, C), jnp.float32),
            pltpu.VMEM((2, CH, OC), jnp.float32),
            pltpu.SemaphoreType.DMA((2,)),
            pltpu.SemaphoreType.DMA((2,)),
            pltpu.SemaphoreType.DMA((2,)),
        ],
        compiler_params=pltpu.CompilerParams(use_tc_tiling_on_sc=False),
    )


@jax.jit
def kernel(input, grid):
    sc_call = _make_sc_call()
    outs = []
    for n in range(N):
        table = jnp.transpose(input[n], (1, 2, 0)).reshape(P, C)
        gx = grid[n, :, :, 0].reshape(P)
        gy = grid[n, :, :, 1].reshape(P)
        rows = sc_call(table, gx, gy)
        outs.append(jnp.transpose(rows.reshape(H, W, C), (2, 0, 1)))
    return jnp.stack(outs)


# speedup vs baseline: 1.2473x; 1.0175x over previous
import jax
import jax.numpy as jnp
from jax import lax
from jax.experimental import pallas as pl
from jax.experimental.pallas import tpu as pltpu
from jax.experimental.pallas import tpu_sc as plsc

N, C, H, W = 4, 96, 384, 384
P = H * W
NP = N * P
NW = 32
PPW = P // NW
CH = 64
CHUNKS = PPW // CH
NB = 3
G16 = CH // 16
CW = C // 32
TW = C // 2
OC = C


def _sc_body(table_hbm, gx_hbm, gy_hbm, out_hbm,
             gx_v, gy_v, idx_v, w_v, r_v, out_v,
             sem_gr, sem_g, sem_o):
    cid = lax.axis_index("c")
    sid = lax.axis_index("s")
    wid = sid * 2 + cid
    base = wid * PPW

    def start_grid(k, b):
        off = base + k * CH
        pltpu.make_async_copy(gx_hbm.at[pl.ds(off, CH)], gx_v.at[b], sem_gr.at[b]).start()
        pltpu.make_async_copy(gy_hbm.at[pl.ds(off, CH)], gy_v.at[b], sem_gr.at[b]).start()

    def wait_grid(k, b):
        off = base + k * CH
        pltpu.make_async_copy(gx_hbm.at[pl.ds(off, CH)], gx_v.at[b], sem_gr.at[b]).wait()
        pltpu.make_async_copy(gy_hbm.at[pl.ds(off, CH)], gy_v.at[b], sem_gr.at[b]).wait()

    def idx_compute(b):
        def idx_body(g, c2):
            s = pl.ds(g * 16, 16)
            x = gx_v[b, s]
            y = gy_v[b, s]
            ix = ((x + 1.0) * W - 1.0) * 0.5
            iy = ((y + 1.0) * H - 1.0) * 0.5
            ixt = ix.astype(jnp.int32)
            ixtf = ixt.astype(jnp.float32)
            mx = ix < ixtf
            ix0 = ixt - jnp.where(mx, 1, 0)
            fx0 = ixtf - jnp.where(mx, 1.0, 0.0)
            iyt = iy.astype(jnp.int32)
            iytf = iyt.astype(jnp.float32)
            my = iy < iytf
            iy0 = iyt - jnp.where(my, 1, 0)
            fy0 = iytf - jnp.where(my, 1.0, 0.0)
            wx1 = ix - fx0
            wx0 = 1.0 - wx1
            wy1 = iy - fy0
            wy0 = 1.0 - wy1
            vx0 = (ix0 >= 0) & (ix0 <= W - 1)
            vx1 = (ix0 >= -1) & (ix0 <= W - 2)
            vy0 = (iy0 >= 0) & (iy0 <= H - 1)
            vy1 = (iy0 >= -1) & (iy0 <= H - 2)
            wx0 = jnp.where(vx0, wx0, 0.0)
            wx1 = jnp.where(vx1, wx1, 0.0)
            wy0 = jnp.where(vy0, wy0, 0.0)
            wy1 = jnp.where(vy1, wy1, 0.0)
            cx0 = jnp.minimum(jnp.maximum(ix0, 0), W - 1)
            cx1 = jnp.minimum(jnp.maximum(ix0 + 1, 0), W - 1)
            cy0 = jnp.minimum(jnp.maximum(iy0, 0), H - 1)
            cy1 = jnp.minimum(jnp.maximum(iy0 + 1, 0), H - 1)
            rb0 = cy0 * W
            rb1 = cy1 * W
            idx_v[b, 0, s] = rb0 + cx0
            idx_v[b, 1, s] = rb0 + cx1
            idx_v[b, 2, s] = rb1 + cx0
            idx_v[b, 3, s] = rb1 + cx1
            w_v[b, 0, s] = wy0 * wx0
            w_v[b, 1, s] = wy0 * wx1
            w_v[b, 2, s] = wy1 * wx0
            w_v[b, 3, s] = wy1 * wx1
            return c2

        lax.fori_loop(0, G16, idx_body, 0)

    def start_gathers(b):
        for q in range(4):
            pltpu.make_async_copy(table_hbm.at[idx_v.at[b, q]], r_v.at[b, q],
                                  sem_g.at[b]).start()

    def wait_gathers(b):
        for q in range(4):
            pltpu.make_async_copy(table_hbm.at[idx_v.at[b, q]], r_v.at[b, q],
                                  sem_g.at[b]).wait()

    def combine(b):
        def cmb_body(g, c2):
            s = pl.ds(g * 16, 16)
            w00g = w_v[b, 0, s]
            w01g = w_v[b, 1, s]
            w10g = w_v[b, 2, s]
            w11g = w_v[b, 3, s]
            p0 = g * 16
            for i in range(16):
                px = p0 + i
                ws = (w00g[i], w01g[i], w10g[i], w11g[i])
                for j in range(C // 16):
                    cs = pl.ds(j * 16, 16)
                    acc = (r_v[b, 0, px, cs] * ws[0]
                           + r_v[b, 1, px, cs] * ws[1]
                           + r_v[b, 2, px, cs] * ws[2]
                           + r_v[b, 3, px, cs] * ws[3])
                    out_v[b, px, cs] = acc
            return c2

        lax.fori_loop(0, G16, cmb_body, 0)

    def start_out(k, b):
        off = base + k * CH
        pltpu.make_async_copy(out_v.at[b], out_hbm.at[pl.ds(off, CH)], sem_o.at[b]).start()

    def wait_out(k, b):
        off = base + k * CH
        pltpu.make_async_copy(out_v.at[b], out_hbm.at[pl.ds(off, CH)], sem_o.at[b]).wait()

    def step(k, b):
        b2 = (b + 2) % NB

        def prefetch():
            wait_grid(k + 2, b2)
            idx_compute(b2)
            start_gathers(b2)

        pl.when(k + 2 < CHUNKS)(prefetch)
        pl.when(k + 3 < CHUNKS)(lambda: start_grid(k + 3, b))
        wait_gathers(b)
        pl.when(k >= NB)(lambda: wait_out(k - NB, b))
        combine(b)
        start_out(k, b)

    start_grid(0, 0)
    start_grid(1, 1)
    wait_grid(0, 0)
    idx_compute(0)
    start_gathers(0)
    wait_grid(1, 1)
    idx_compute(1)
    start_gathers(1)
    start_grid(2, 2)

    def loop_body(k3, carry):
        step(3 * k3, 0)
        step(3 * k3 + 1, 1)
        step(3 * k3 + 2, 2)
        return carry

    lax.fori_loop(0, CHUNKS // NB, loop_body, 0)

    wait_out(CHUNKS - 3, 0)
    wait_out(CHUNKS - 2, 1)
    wait_out(CHUNKS - 1, 2)


def _make_sc_call():
    mesh = plsc.VectorSubcoreMesh(core_axis_name="c", subcore_axis_name="s")
    return pl.kernel(
        _sc_body,
        out_type=jax.ShapeDtypeStruct((P, OC), jnp.float32),
        mesh=mesh,
        scratch_types=[
            pltpu.VMEM((NB, CH), jnp.float32),
            pltpu.VMEM((NB, CH), jnp.float32),
            pltpu.VMEM((NB, 4, CH), jnp.int32),
            pltpu.VMEM((NB, 4, CH), jnp.float32),
            pltpu.VMEM((NB, 4, CH, C), jnp.float32),
            pltpu.VMEM((NB, CH, OC), jnp.float32),
            pltpu.SemaphoreType.DMA((NB,)),
            pltpu.SemaphoreType.DMA((NB,)),
            pltpu.SemaphoreType.DMA((NB,)),
        ],
        compiler_params=pltpu.CompilerParams(use_tc_tiling_on_sc=False),
    )


@jax.jit
def kernel(input, grid):
    sc_call = _make_sc_call()
    outs = []
    for n in range(N):
        table = jnp.transpose(input[n], (1, 2, 0)).reshape(P, C)
        gx = grid[n, :, :, 0].reshape(P)
        gy = grid[n, :, :, 1].reshape(P)
        rows = sc_call(table, gx, gy)
        outs.append(jnp.transpose(rows.reshape(H, W, C), (2, 0, 1)))
    return jnp.stack(outs)
